# baseline (speedup 1.0000x reference)
import jax
import jax.numpy as jnp
from jax.experimental import pallas as pl
from jax.experimental.pallas import tpu as pltpu

_SQRT_2_OVER_PI = 0.7978845608028654


def _make_body(NA, BF):
    def _mlp_body(x_ref, w1_ref, b1_ref, w2_ref, b2_ref, o_ref, h_ref):
        s = pl.program_id(1)

        @pl.when(s < NA)
        def _():
            h = jax.lax.dot_general(
                x_ref[...], w1_ref[...], (((1,), (1,)), ((), ())),
                preferred_element_type=jnp.float32)
            h = h + b1_ref[...]
            h = 0.5 * h * (1.0 + jnp.tanh(
                _SQRT_2_OVER_PI * (h + 0.044715 * h * h * h)))
            h_ref[:, pl.ds(s * BF, BF)] = h

        @pl.when(s >= NA)
        def _():
            p = jax.lax.dot_general(
                h_ref[...], w2_ref[...], (((1,), (1,)), ((), ())),
                preferred_element_type=jnp.float32)
            o_ref[...] = p + b2_ref[...]

    return _mlp_body


def kernel(x, W1, b1, W2, b2, Wg):
    B, S, D = x.shape
    M = B * S
    FF = W1.shape[0]
    xf = x.reshape(M, D)
    BM = min(512, M)
    BF = min(512, FF)
    BD = min(256, D)
    NA = FF // BF
    ND = D // BD
    grid = (M // BM, NA + ND)
    out = pl.pallas_call(
        _make_body(NA, BF),
        grid=grid,
        in_specs=[
            pl.BlockSpec((BM, D), lambda i, s: (i, 0)),
            pl.BlockSpec((BF, D), lambda i, s: (jnp.minimum(s, NA - 1), 0)),
            pl.BlockSpec((1, BF), lambda i, s: (0, jnp.minimum(s, NA - 1))),
            pl.BlockSpec((BD, FF), lambda i, s: (jnp.maximum(s - NA, 0), 0)),
            pl.BlockSpec((1, BD), lambda i, s: (0, jnp.maximum(s - NA, 0))),
        ],
        out_specs=pl.BlockSpec((BM, BD), lambda i, s: (i, jnp.maximum(s - NA, 0))),
        out_shape=jax.ShapeDtypeStruct((M, D), jnp.float32),
        scratch_shapes=[pltpu.VMEM((BM, FF), jnp.float32)],
        compiler_params=pltpu.CompilerParams(
            dimension_semantics=("parallel", "arbitrary"),
            vmem_limit_bytes=63 * 1024 * 1024,
        ),
    )(xf, W1, b1.reshape(1, FF), W2, b2.reshape(1, D))
    return out.reshape(B, S, D)


# speedup vs baseline: 1.1842x; 1.1842x over previous
import jax
import jax.numpy as jnp
from jax.experimental import pallas as pl
from jax.experimental.pallas import tpu as pltpu

_SQRT_2_OVER_PI = 0.7978845608028654


def _make_body(NA, BF):
    def _mlp_body(x_ref, w1_ref, b1_ref, w2_ref, b2_ref, o_ref, h_ref):
        s = pl.program_id(1)

        @pl.when(s < NA)
        def _():
            h = jax.lax.dot_general(
                x_ref[...], w1_ref[...], (((1,), (1,)), ((), ())),
                preferred_element_type=jnp.float32)
            h = h + b1_ref[...]
            h = 0.5 * h * (1.0 + jnp.tanh(
                _SQRT_2_OVER_PI * (h + 0.044715 * h * h * h)))
            h_ref[:, pl.ds(s * BF, BF)] = h.astype(h_ref.dtype)

        @pl.when(s >= NA)
        def _():
            p = jax.lax.dot_general(
                h_ref[...], w2_ref[...], (((1,), (1,)), ((), ())),
                preferred_element_type=jnp.float32)
            o_ref[...] = p + b2_ref[...]

    return _mlp_body


def kernel(x, W1, b1, W2, b2, Wg):
    B, S, D = x.shape
    M = B * S
    FF = W1.shape[0]
    xf = x.reshape(M, D).astype(jnp.bfloat16)
    W1 = W1.astype(jnp.bfloat16)
    W2 = W2.astype(jnp.bfloat16)
    BM = min(1024, M)
    BF = min(512, FF)
    BD = min(256, D)
    NA = FF // BF
    ND = D // BD
    grid = (M // BM, NA + ND)
    out = pl.pallas_call(
        _make_body(NA, BF),
        grid=grid,
        in_specs=[
            pl.BlockSpec((BM, D), lambda i, s: (i, 0)),
            pl.BlockSpec((BF, D), lambda i, s: (jnp.minimum(s, NA - 1), 0)),
            pl.BlockSpec((1, BF), lambda i, s: (0, jnp.minimum(s, NA - 1))),
            pl.BlockSpec((BD, FF), lambda i, s: (jnp.maximum(s - NA, 0), 0)),
            pl.BlockSpec((1, BD), lambda i, s: (0, jnp.maximum(s - NA, 0))),
        ],
        out_specs=pl.BlockSpec((BM, BD), lambda i, s: (i, jnp.maximum(s - NA, 0))),
        out_shape=jax.ShapeDtypeStruct((M, D), jnp.float32),
        scratch_shapes=[pltpu.VMEM((BM, FF), jnp.bfloat16)],
        compiler_params=pltpu.CompilerParams(
            dimension_semantics=("parallel", "arbitrary"),
            vmem_limit_bytes=63 * 1024 * 1024,
        ),
    )(xf, W1, b1.reshape(1, FF), W2, b2.reshape(1, D))
    return out.reshape(B, S, D)
